# baseline (speedup 1.0000x reference)
import jax
import jax.numpy as jnp
from jax.experimental import pallas as pl
from jax.experimental.pallas import tpu as pltpu
from jax.experimental.pallas import tpu_sc as plsc

_BR = 32
_BC = 128


def kernel(x, pos_emb):
    B, T, D = x.shape
    N = T * D
    xt = x.transpose(1, 2, 0).reshape(N, B)
    pe_b = jnp.broadcast_to(pos_emb.reshape(N)[:, None], (N, _BC))

    mesh = plsc.VectorSubcoreMesh(core_axis_name="core",
                                  subcore_axis_name="subcore")

    @pl.kernel(out_type=jax.ShapeDtypeStruct((N, B), x.dtype), mesh=mesh)
    def sc_add(x_hbm, pe_hbm, o_hbm):
        def body(x_vmem, pe_vmem, o_vmem):
            @pl.loop(0, _BR)
            def _(r):
                @pl.loop(0, _BC, step=16)
                def _(c):
                    slc = (pl.ds(r, 1), pl.ds(c, 16))
                    o_vmem.at[*slc][...] = (
                        x_vmem.at[*slc][...] + pe_vmem.at[*slc][...]
                    )

        pltpu.emit_pipeline(
            body,
            grid=(N // _BR, B // _BC),
            in_specs=[
                pl.BlockSpec((_BR, _BC), index_map=lambda i, j: (i, j)),
                pl.BlockSpec((_BR, _BC), index_map=lambda i, j: (i, 0)),
            ],
            out_specs=[pl.BlockSpec((_BR, _BC), index_map=lambda i, j: (i, j))],
            core_axis_name=("core", "subcore"),
            dimension_semantics=(pltpu.PARALLEL, pltpu.PARALLEL),
        )(x_hbm, pe_hbm, o_hbm)

    out = sc_add(xt, pe_b)
    return out.reshape(T, D, B).transpose(2, 0, 1)


# speedup vs baseline: 4.8665x; 4.8665x over previous
import jax
import jax.numpy as jnp
from jax.experimental import pallas as pl

_RR = 640


def _add_kernel(x_ref, pe_ref, o_ref):
    pev = pe_ref[0]
    g = _RR // 128
    rep = jnp.broadcast_to(pev[:, None, :], (g, 128, 128)).reshape(_RR, 128)
    sub = jax.lax.broadcasted_iota(jnp.int32, (_RR, 128), 0) % 128
    lane = jax.lax.broadcasted_iota(jnp.int32, (_RR, 128), 1)
    pe_col = jnp.sum(jnp.where(sub == lane, rep, 0.0), axis=1, keepdims=True)
    o_ref[0] = x_ref[0] + pe_col


def kernel(x, pos_emb):
    B, T, D = x.shape
    N = T * D
    G = N // _RR
    xt = x.transpose(1, 2, 0).reshape(G, _RR, B)
    pe = pos_emb.reshape(G, _RR // 128, 128)
    out = pl.pallas_call(
        _add_kernel,
        grid=(G,),
        in_specs=[
            pl.BlockSpec((1, _RR, B), lambda i: (i, 0, 0)),
            pl.BlockSpec((1, _RR // 128, 128), lambda i: (i, 0, 0)),
        ],
        out_specs=pl.BlockSpec((1, _RR, B), lambda i: (i, 0, 0)),
        out_shape=jax.ShapeDtypeStruct((G, _RR, B), x.dtype),
    )(xt, pe)
    return out.reshape(T, D, B).transpose(2, 0, 1)


# speedup vs baseline: 4.9273x; 1.0125x over previous
import jax
import jax.numpy as jnp
from jax.experimental import pallas as pl

_RR = 640


def _make_kernel(T, D):
    U = _RR // D

    def _add_kernel(x_ref, pe_ref, o_ref):
        i = pl.program_id(0)
        peT = pe_ref[...]
        rep = jnp.broadcast_to(peT[None], (U, D, T)).reshape(_RR, T)
        sub = jax.lax.broadcasted_iota(jnp.int32, (_RR, T), 0)
        lane = jax.lax.broadcasted_iota(jnp.int32, (_RR, T), 1)
        mask = lane == (U * i + sub // D)
        pe_col = jnp.sum(jnp.where(mask, rep, 0.0), axis=1, keepdims=True)
        o_ref[0] = x_ref[0] + pe_col

    return _add_kernel


def kernel(x, pos_emb):
    B, T, D = x.shape
    N = T * D
    G = N // _RR
    xt = x.transpose(1, 2, 0).reshape(G, _RR, B)
    out = pl.pallas_call(
        _make_kernel(T, D),
        grid=(G,),
        in_specs=[
            pl.BlockSpec((1, _RR, B), lambda i: (i, 0, 0)),
            pl.BlockSpec((D, T), lambda i: (0, 0)),
        ],
        out_specs=pl.BlockSpec((1, _RR, B), lambda i: (i, 0, 0)),
        out_shape=jax.ShapeDtypeStruct((G, _RR, B), x.dtype),
    )(xt, pos_emb.T)
    return out.reshape(T, D, B).transpose(2, 0, 1)
